# baseline (speedup 1.0000x reference)
import functools

import jax
import jax.numpy as jnp
from jax import lax
from jax.experimental import pallas as pl
from jax.experimental.pallas import tpu as pltpu
from jax.experimental.pallas import tpu_sc as plsc

_NUM_ROUTE_COMMANDS = 8
_OUTPUT_DIM = 128
_BATCH = 16384


@functools.lru_cache(maxsize=None)
def _make_gather(batch: int, dim: int):
    info = plsc.get_sparse_core_info()
    nw = info.num_cores * info.num_subcores
    assert batch % (8 * nw) == 0
    b_per_w = batch // nw
    mesh = plsc.VectorSubcoreMesh(core_axis_name="c", subcore_axis_name="s")

    chunk = 64
    nch = b_per_w // chunk
    nbuf = 3
    assert nch >= nbuf

    @functools.partial(
        pl.kernel,
        mesh=mesh,
        out_type=[
            jax.ShapeDtypeStruct((batch, dim), jnp.float32),
            jax.ShapeDtypeStruct((batch,), jnp.int32),
        ],
        scratch_types=[
            pltpu.VMEM((b_per_w,), jnp.int32),
            pltpu.VMEM((3, chunk, dim), jnp.float32),
            pltpu.VMEM_SHARED((info.num_subcores, _NUM_ROUTE_COMMANDS, dim),
                              jnp.float32),
            pltpu.SemaphoreType.DMA,
            pltpu.SemaphoreType.DMA,
            pltpu.SemaphoreType.DMA,
        ],
    )
    def gather(ids_hbm, table_hbm, out_hbm, ids_out_hbm,
               idx_v, rows_v, table_sh, gsem, wsem, msem):
        sid = lax.axis_index("s")
        wid = sid * info.num_cores + lax.axis_index("c")
        base = wid * b_per_w

        icp = pltpu.async_copy(ids_hbm.at[pl.ds(base, b_per_w)], idx_v, wsem)
        tcp = pltpu.async_copy(table_hbm, table_sh.at[sid], gsem)
        icp.wait()
        tcp.wait()
        my_table = table_sh.at[sid]

        def start_gather(c):
            return pltpu.async_copy(
                my_table.at[idx_v.at[pl.ds(c * chunk, chunk)]],
                rows_v.at[c % nbuf], gsem)

        g = [None] * nch
        writes = [None] * nch
        g[0] = start_gather(0)
        g[1] = start_gather(1)

        mcp = pltpu.async_copy(idx_v, ids_out_hbm.at[pl.ds(base, b_per_w)], msem)

        for c in range(nch):
            g[c].wait()
            if c >= nbuf - 1:
                writes[c - (nbuf - 1)].wait()
            if c + 2 < nch:
                g[c + 2] = start_gather(c + 2)
            writes[c] = pltpu.async_copy(
                rows_v.at[c % nbuf],
                out_hbm.at[pl.ds(base + c * chunk, chunk)], wsem)
        writes[nch - 2].wait()
        writes[nch - 1].wait()
        mcp.wait()

    return gather


def kernel(route_command_ids, embedding_weight):
    batch = route_command_ids.shape[0]
    dim = embedding_weight.shape[1]
    rows, ids = _make_gather(batch, dim)(
        route_command_ids.astype(jnp.int32), embedding_weight)
    tokens = rows.reshape(batch, 1, dim)
    valid_mask = jnp.ones((batch, 1), dtype=jnp.bool_)
    return (tokens, valid_mask, ids)


# speedup vs baseline: 1.0336x; 1.0336x over previous
import functools

import jax
import jax.numpy as jnp
from jax import lax
from jax.experimental import pallas as pl
from jax.experimental.pallas import tpu as pltpu
from jax.experimental.pallas import tpu_sc as plsc

_NUM_ROUTE_COMMANDS = 8
_OUTPUT_DIM = 128
_BATCH = 16384


@functools.lru_cache(maxsize=None)
def _make_gather(batch: int, dim: int):
    info = plsc.get_sparse_core_info()
    nw = info.num_cores * info.num_subcores
    assert batch % (8 * nw) == 0
    b_per_w = batch // nw
    mesh = plsc.VectorSubcoreMesh(core_axis_name="c", subcore_axis_name="s")

    chunk = 64
    nch = b_per_w // chunk
    nbuf = 3
    assert nch >= nbuf

    @functools.partial(
        pl.kernel,
        mesh=mesh,
        out_type=[
            jax.ShapeDtypeStruct((batch, dim), jnp.float32),
            jax.ShapeDtypeStruct((batch,), jnp.int32),
        ],
        scratch_types=[
            pltpu.VMEM((b_per_w,), jnp.int32),
            pltpu.VMEM((3, chunk, dim), jnp.float32),
            pltpu.VMEM_SHARED((_NUM_ROUTE_COMMANDS, dim), jnp.float32),
            pltpu.SemaphoreType.DMA,
            pltpu.SemaphoreType.DMA,
            pltpu.SemaphoreType.DMA,
        ],
    )
    def gather(ids_hbm, table_hbm, out_hbm, ids_out_hbm,
               idx_v, rows_v, table_sh, gsem, wsem, msem):
        sid = lax.axis_index("s")
        wid = sid * info.num_cores + lax.axis_index("c")
        base = wid * b_per_w

        icp = pltpu.async_copy(ids_hbm.at[pl.ds(base, b_per_w)], idx_v, wsem)

        @pl.when(sid == 0)
        def _():
            pltpu.sync_copy(table_hbm, table_sh)

        icp.wait()

        mcp = pltpu.async_copy(idx_v, ids_out_hbm.at[pl.ds(base, b_per_w)], msem)

        plsc.subcore_barrier()

        def start_gather(c):
            return pltpu.async_copy(
                table_sh.at[idx_v.at[pl.ds(c * chunk, chunk)]],
                rows_v.at[c % nbuf], gsem)

        g = [None] * nch
        writes = [None] * nch
        g[0] = start_gather(0)
        g[1] = start_gather(1)
        for c in range(nch):
            g[c].wait()
            if c >= nbuf - 1:
                writes[c - (nbuf - 1)].wait()
            if c + 2 < nch:
                g[c + 2] = start_gather(c + 2)
            writes[c] = pltpu.async_copy(
                rows_v.at[c % nbuf],
                out_hbm.at[pl.ds(base + c * chunk, chunk)], wsem)
        writes[nch - 2].wait()
        writes[nch - 1].wait()
        mcp.wait()

    return gather


def kernel(route_command_ids, embedding_weight):
    batch = route_command_ids.shape[0]
    dim = embedding_weight.shape[1]
    rows, ids = _make_gather(batch, dim)(
        route_command_ids.astype(jnp.int32), embedding_weight)
    tokens = rows.reshape(batch, 1, dim)
    valid_mask = jnp.ones((batch, 1), dtype=jnp.bool_)
    return (tokens, valid_mask, ids)


# speedup vs baseline: 1.0391x; 1.0054x over previous
import functools

import jax
import jax.numpy as jnp
from jax import lax
from jax.experimental import pallas as pl
from jax.experimental.pallas import tpu as pltpu
from jax.experimental.pallas import tpu_sc as plsc

_NUM_ROUTE_COMMANDS = 8
_OUTPUT_DIM = 128
_BATCH = 16384


@functools.lru_cache(maxsize=None)
def _make_gather(batch: int, dim: int):
    info = plsc.get_sparse_core_info()
    nw = info.num_cores * info.num_subcores
    assert batch % (8 * nw) == 0
    b_per_w = batch // nw
    mesh = plsc.VectorSubcoreMesh(core_axis_name="c", subcore_axis_name="s")

    chunk = 128
    nch = b_per_w // chunk
    nbuf = 3
    assert nch >= nbuf

    @functools.partial(
        pl.kernel,
        mesh=mesh,
        out_type=[
            jax.ShapeDtypeStruct((batch, dim), jnp.float32),
            jax.ShapeDtypeStruct((batch,), jnp.int32),
        ],
        scratch_types=[
            pltpu.VMEM((b_per_w,), jnp.int32),
            pltpu.VMEM((nbuf, chunk, dim), jnp.float32),
            pltpu.VMEM_SHARED((_NUM_ROUTE_COMMANDS, dim), jnp.float32),
            pltpu.SemaphoreType.DMA,
            pltpu.SemaphoreType.DMA,
            pltpu.SemaphoreType.DMA,
        ],
    )
    def gather(ids_hbm, table_hbm, out_hbm, ids_out_hbm,
               idx_v, rows_v, table_sh, gsem, wsem, msem):
        sid = lax.axis_index("s")
        wid = sid * info.num_cores + lax.axis_index("c")
        base = wid * b_per_w

        icp = pltpu.async_copy(ids_hbm.at[pl.ds(base, b_per_w)], idx_v, wsem)

        @pl.when(sid == 0)
        def _():
            pltpu.sync_copy(table_hbm, table_sh)

        icp.wait()

        mcp = pltpu.async_copy(idx_v, ids_out_hbm.at[pl.ds(base, b_per_w)], msem)

        plsc.subcore_barrier()

        def start_gather(c):
            return pltpu.async_copy(
                table_sh.at[idx_v.at[pl.ds(c * chunk, chunk)]],
                rows_v.at[c % nbuf], gsem)

        g = [None] * nch
        writes = [None] * nch
        g[0] = start_gather(0)
        g[1] = start_gather(1)
        for c in range(nch):
            g[c].wait()
            if c >= nbuf - 1:
                writes[c - (nbuf - 1)].wait()
            if c + 2 < nch:
                g[c + 2] = start_gather(c + 2)
            writes[c] = pltpu.async_copy(
                rows_v.at[c % nbuf],
                out_hbm.at[pl.ds(base + c * chunk, chunk)], wsem)
        writes[nch - 2].wait()
        writes[nch - 1].wait()
        mcp.wait()

    return gather


def kernel(route_command_ids, embedding_weight):
    batch = route_command_ids.shape[0]
    dim = embedding_weight.shape[1]
    rows, ids = _make_gather(batch, dim)(
        route_command_ids.astype(jnp.int32), embedding_weight)
    tokens = rows.reshape(batch, 1, dim)
    valid_mask = jnp.ones((batch, 1), dtype=jnp.bool_)
    return (tokens, valid_mask, ids)
